# baseline (speedup 1.0000x reference)
import functools

import jax
import jax.numpy as jnp
from jax import lax
from jax.experimental import pallas as pl
from jax.experimental.pallas import tpu as pltpu
from jax.experimental.pallas import tpu_sc as plsc

N_NODES = 10000
N_EDGES = 320000
DIM = 128

NC = 2
NS = 16
NW = NC * NS
CHUNK = 128
E_PAD = ((N_EDGES + NW * CHUNK - 1) // (NW * CHUNK)) * (NW * CHUNK)
NCHUNK0 = 108
NCHUNK1 = 50
MAXC = max(NCHUNK0, NCHUNK1)
assert NS * (NCHUNK0 + NCHUNK1) * CHUNK == E_PAD
N_PAD = 10240
RPT = N_PAD // NS


def _spmm_body(h_hbm, src_hbm, dst_hbm, w_hbm, zeros_hbm, out_hbm,
               src_v, rb0, rb1, db0, db1, wb0, wb1, acc_sh, g0, g1, s0, s1):
    c = lax.axis_index("c")
    s = lax.axis_index("s")
    wid = c * NS + s
    npair = jnp.where(c == 0, NCHUNK0 // 2, NCHUNK1 // 2)

    pltpu.sync_copy(src_hbm.at[wid], src_v)

    def scale(rb, wb):
        def group_body(g, carry):
            wv16 = wb[pl.ds(g * 16, 16)]
            for t in range(16):
                e = g * 16 + t
                ws = wv16[t]
                for k in range(DIM // 16):
                    sl = pl.ds(k * 16, 16)
                    rb[e, sl] = rb[e, sl] * ws
            return carry

        lax.fori_loop(0, CHUNK // 16, group_body, 0)

    def fetch(j, rb, db, wb, sem):
        off = (wid * MAXC + j) * CHUNK
        pltpu.async_copy(h_hbm.at[src_v.at[j]], rb, sem)
        pltpu.async_copy(dst_hbm.at[pl.ds(off, CHUNK)], db, sem)
        pltpu.async_copy(w_hbm.at[pl.ds(off, CHUNK)], wb, sem)

    def wait_fetch(j, rb, db, wb, sem):
        off = (wid * MAXC + j) * CHUNK
        pltpu.make_async_copy(h_hbm.at[src_v.at[j]], rb, sem).wait()
        pltpu.make_async_copy(dst_hbm.at[pl.ds(off, CHUNK)], db, sem).wait()
        pltpu.make_async_copy(w_hbm.at[pl.ds(off, CHUNK)], wb, sem).wait()

    def scatter(rb, db, sem):
        pltpu.async_copy(rb, acc_sh.at[db], sem, add=True)

    def wait_scatter(rb, db, sem):
        pltpu.make_async_copy(rb, acc_sh.at[db], sem).wait()

    fetch(0, rb0, db0, wb0, g0)
    pltpu.sync_copy(zeros_hbm, acc_sh.at[pl.ds(s * RPT, RPT)])
    plsc.subcore_barrier()

    def pair_body(jj, carry):
        j0 = jj * 2
        j1 = j0 + 1

        @pl.when(jj > 0)
        def _():
            wait_scatter(rb1, db1, s1)

        fetch(j1, rb1, db1, wb1, g1)
        wait_fetch(j0, rb0, db0, wb0, g0)
        scale(rb0, wb0)
        scatter(rb0, db0, s0)
        wait_fetch(j1, rb1, db1, wb1, g1)
        scale(rb1, wb1)
        wait_scatter(rb0, db0, s0)

        @pl.when(jj + 1 < npair)
        def _():
            fetch(j0 + 2, rb0, db0, wb0, g0)

        scatter(rb1, db1, s1)
        return carry

    lax.fori_loop(0, npair, pair_body, 0)
    wait_scatter(rb1, db1, s1)
    plsc.subcore_barrier()
    pltpu.sync_copy(acc_sh.at[pl.ds(s * RPT, RPT)],
                    out_hbm.at[c, pl.ds(s * RPT, RPT)])


_spmm_sc = functools.partial(
    pl.kernel,
    out_type=jax.ShapeDtypeStruct((NC, N_PAD, DIM), jnp.float32),
    mesh=plsc.VectorSubcoreMesh(core_axis_name="c", subcore_axis_name="s"),
    scratch_types=[
        pltpu.VMEM((MAXC, CHUNK), jnp.int32),
        pltpu.VMEM((CHUNK, DIM), jnp.float32),
        pltpu.VMEM((CHUNK, DIM), jnp.float32),
        pltpu.VMEM((CHUNK,), jnp.int32),
        pltpu.VMEM((CHUNK,), jnp.int32),
        pltpu.VMEM((CHUNK,), jnp.float32),
        pltpu.VMEM((CHUNK,), jnp.float32),
        pltpu.VMEM_SHARED((N_PAD, DIM), jnp.float32),
        pltpu.SemaphoreType.DMA,
        pltpu.SemaphoreType.DMA,
        pltpu.SemaphoreType.DMA,
        pltpu.SemaphoreType.DMA,
    ],
)(_spmm_body)


_BLK = 2000


def _li0_tc(x_ref, w_ref, b_ref, o_ref):
    o_ref[...] = (
        jnp.dot(x_ref[...], w_ref[...], preferred_element_type=jnp.float32)
        + b_ref[...])


def _add_tc(a_ref, b_ref, o_ref):
    o_ref[...] = a_ref[...] + b_ref[...]


def _li1_tc(a_ref, b_ref, w_ref, bias_ref, o_ref):
    h = jnp.maximum(a_ref[...] + b_ref[...], 0.0)
    o_ref[...] = (
        jnp.dot(h, w_ref[...], preferred_element_type=jnp.float32)
        + bias_ref[...])


def _row_spec():
    return pl.BlockSpec((_BLK, DIM), lambda i: (i, 0))


def _full_spec(shape):
    return pl.BlockSpec(shape, lambda i: (0,) * len(shape))


def _li0(x, W0, b0):
    return pl.pallas_call(
        _li0_tc,
        grid=(N_NODES // _BLK,),
        in_specs=[_row_spec(), _full_spec((DIM, DIM)), _full_spec((1, DIM))],
        out_specs=_row_spec(),
        out_shape=jax.ShapeDtypeStruct((N_NODES, DIM), jnp.float32),
    )(x, W0, b0.reshape(1, DIM))


def _add(p):
    return pl.pallas_call(
        _add_tc,
        grid=(N_NODES // _BLK,),
        in_specs=[_row_spec(), _row_spec()],
        out_specs=_row_spec(),
        out_shape=jax.ShapeDtypeStruct((N_NODES, DIM), jnp.float32),
    )(p[0], p[1])


def _li1(q, W1, b1):
    return pl.pallas_call(
        _li1_tc,
        grid=(N_NODES // _BLK,),
        in_specs=[_row_spec(), _row_spec(), _full_spec((DIM, DIM)),
                  _full_spec((1, DIM))],
        out_specs=_row_spec(),
        out_shape=jax.ShapeDtypeStruct((N_NODES, DIM), jnp.float32),
    )(q[0], q[1], W1, b1.reshape(1, DIM))


def _layout(flat, dtype):
    e0 = NS * NCHUNK0 * CHUNK
    part0 = flat[:e0].reshape(NS, NCHUNK0, CHUNK)
    part1 = flat[e0:].reshape(NS, NCHUNK1, CHUNK)
    arr = jnp.zeros((NW, MAXC, CHUNK), dtype)
    arr = arr.at[:NS, :NCHUNK0].set(part0)
    arr = arr.at[NS:, :NCHUNK1].set(part1)
    return arr


def kernel(x, edge_index, edge_weight, W0, b0, W1, b1):
    pad = E_PAD - N_EDGES
    src = _layout(jnp.pad(edge_index[0].astype(jnp.int32), (0, pad)),
                  jnp.int32)
    dst = _layout(jnp.pad(edge_index[1].astype(jnp.int32), (0, pad)),
                  jnp.int32).reshape(-1)
    w = _layout(jnp.pad(edge_weight.astype(jnp.float32), (0, pad)),
                jnp.float32).reshape(-1)
    zeros = jnp.zeros((RPT, DIM), jnp.float32)

    h0 = _li0(x, W0, b0)
    p = _spmm_sc(h0, src, dst, w, zeros)
    h1 = _add(p)
    q = _spmm_sc(h1, src, dst, w, zeros)
    return _li1(q, W1, b1)


# speedup vs baseline: 1.0581x; 1.0581x over previous
import functools

import jax
import jax.numpy as jnp
from jax import lax
from jax.experimental import pallas as pl
from jax.experimental.pallas import tpu as pltpu
from jax.experimental.pallas import tpu_sc as plsc

N_NODES = 10000
N_EDGES = 320000
DIM = 128

NC = 2
NS = 16
NW = NC * NS
CHUNK = 128
E_PAD = ((N_EDGES + NW * CHUNK - 1) // (NW * CHUNK)) * (NW * CHUNK)
NCHUNK0 = 104
NCHUNK1 = 54
MAXC = max(NCHUNK0, NCHUNK1)
assert NS * (NCHUNK0 + NCHUNK1) * CHUNK == E_PAD
N_PAD = 10240
RPT = N_PAD // NS


def _spmm_body(h_hbm, src_hbm, dst_hbm, w_hbm, zeros_hbm, out_hbm,
               src_v, rb0, rb1, db0, db1, wb0, wb1, acc_sh, g0, g1, s0, s1):
    c = lax.axis_index("c")
    s = lax.axis_index("s")
    wid = c * NS + s
    npair = jnp.where(c == 0, NCHUNK0 // 2, NCHUNK1 // 2)

    pltpu.sync_copy(src_hbm.at[wid], src_v)

    def scale(rb, wb):
        def group_body(g, carry):
            wv16 = wb[pl.ds(g * 16, 16)]
            for t in range(16):
                e = g * 16 + t
                ws = wv16[t]
                for k in range(DIM // 16):
                    sl = pl.ds(k * 16, 16)
                    rb[e, sl] = rb[e, sl] * ws
            return carry

        lax.fori_loop(0, CHUNK // 16, group_body, 0)

    def fetch(j, rb, db, wb, sem):
        off = (wid * MAXC + j) * CHUNK
        pltpu.async_copy(h_hbm.at[src_v.at[j]], rb, sem)
        pltpu.async_copy(dst_hbm.at[pl.ds(off, CHUNK)], db, sem)
        pltpu.async_copy(w_hbm.at[pl.ds(off, CHUNK)], wb, sem)

    def wait_fetch(j, rb, db, wb, sem):
        off = (wid * MAXC + j) * CHUNK
        pltpu.make_async_copy(h_hbm.at[src_v.at[j]], rb, sem).wait()
        pltpu.make_async_copy(dst_hbm.at[pl.ds(off, CHUNK)], db, sem).wait()
        pltpu.make_async_copy(w_hbm.at[pl.ds(off, CHUNK)], wb, sem).wait()

    def scatter(rb, db, sem):
        pltpu.async_copy(rb, acc_sh.at[db], sem, add=True)

    def wait_scatter(rb, db, sem):
        pltpu.make_async_copy(rb, acc_sh.at[db], sem).wait()

    fetch(0, rb0, db0, wb0, g0)
    pltpu.sync_copy(zeros_hbm, acc_sh.at[pl.ds(s * RPT, RPT)])
    plsc.subcore_barrier()

    def pair_body(jj, carry):
        j0 = jj * 2
        j1 = j0 + 1

        @pl.when(jj > 0)
        def _():
            wait_scatter(rb1, db1, s1)

        fetch(j1, rb1, db1, wb1, g1)
        wait_fetch(j0, rb0, db0, wb0, g0)
        scale(rb0, wb0)
        scatter(rb0, db0, s0)
        wait_fetch(j1, rb1, db1, wb1, g1)
        scale(rb1, wb1)
        wait_scatter(rb0, db0, s0)

        @pl.when(jj + 1 < npair)
        def _():
            fetch(j0 + 2, rb0, db0, wb0, g0)

        scatter(rb1, db1, s1)
        return carry

    lax.fori_loop(0, npair, pair_body, 0)
    wait_scatter(rb1, db1, s1)
    plsc.subcore_barrier()
    pltpu.sync_copy(acc_sh.at[pl.ds(s * RPT, RPT)],
                    out_hbm.at[c, pl.ds(s * RPT, RPT)])


_spmm_sc = functools.partial(
    pl.kernel,
    out_type=jax.ShapeDtypeStruct((NC, N_PAD, DIM), jnp.float32),
    mesh=plsc.VectorSubcoreMesh(core_axis_name="c", subcore_axis_name="s"),
    scratch_types=[
        pltpu.VMEM((MAXC, CHUNK), jnp.int32),
        pltpu.VMEM((CHUNK, DIM), jnp.float32),
        pltpu.VMEM((CHUNK, DIM), jnp.float32),
        pltpu.VMEM((CHUNK,), jnp.int32),
        pltpu.VMEM((CHUNK,), jnp.int32),
        pltpu.VMEM((CHUNK,), jnp.float32),
        pltpu.VMEM((CHUNK,), jnp.float32),
        pltpu.VMEM_SHARED((N_PAD, DIM), jnp.float32),
        pltpu.SemaphoreType.DMA,
        pltpu.SemaphoreType.DMA,
        pltpu.SemaphoreType.DMA,
        pltpu.SemaphoreType.DMA,
    ],
)(_spmm_body)


_BLK = 2000


def _li0_tc(x_ref, w_ref, b_ref, o_ref):
    o_ref[...] = (
        jnp.dot(x_ref[...], w_ref[...], preferred_element_type=jnp.float32)
        + b_ref[...])


def _add_tc(a_ref, b_ref, o_ref):
    o_ref[...] = a_ref[...] + b_ref[...]


def _li1_tc(a_ref, b_ref, w_ref, bias_ref, o_ref):
    h = jnp.maximum(a_ref[...] + b_ref[...], 0.0)
    o_ref[...] = (
        jnp.dot(h, w_ref[...], preferred_element_type=jnp.float32)
        + bias_ref[...])


def _row_spec():
    return pl.BlockSpec((_BLK, DIM), lambda i: (i, 0))


def _full_spec(shape):
    return pl.BlockSpec(shape, lambda i: (0,) * len(shape))


def _li0(x, W0, b0):
    return pl.pallas_call(
        _li0_tc,
        grid=(N_NODES // _BLK,),
        in_specs=[_row_spec(), _full_spec((DIM, DIM)), _full_spec((1, DIM))],
        out_specs=_row_spec(),
        out_shape=jax.ShapeDtypeStruct((N_NODES, DIM), jnp.float32),
    )(x, W0, b0.reshape(1, DIM))


def _add(p):
    return pl.pallas_call(
        _add_tc,
        grid=(N_NODES // _BLK,),
        in_specs=[_row_spec(), _row_spec()],
        out_specs=_row_spec(),
        out_shape=jax.ShapeDtypeStruct((N_NODES, DIM), jnp.float32),
    )(p[0], p[1])


def _li1(q, W1, b1):
    return pl.pallas_call(
        _li1_tc,
        grid=(N_NODES // _BLK,),
        in_specs=[_row_spec(), _row_spec(), _full_spec((DIM, DIM)),
                  _full_spec((1, DIM))],
        out_specs=_row_spec(),
        out_shape=jax.ShapeDtypeStruct((N_NODES, DIM), jnp.float32),
    )(q[0], q[1], W1, b1.reshape(1, DIM))


def _layout(flat, dtype):
    e0 = NS * NCHUNK0 * CHUNK
    part0 = flat[:e0].reshape(NS, NCHUNK0, CHUNK)
    part1 = flat[e0:].reshape(NS, NCHUNK1, CHUNK)
    arr = jnp.zeros((NW, MAXC, CHUNK), dtype)
    arr = arr.at[:NS, :NCHUNK0].set(part0)
    arr = arr.at[NS:, :NCHUNK1].set(part1)
    return arr


def kernel(x, edge_index, edge_weight, W0, b0, W1, b1):
    pad = E_PAD - N_EDGES
    src = _layout(jnp.pad(edge_index[0].astype(jnp.int32), (0, pad)),
                  jnp.int32)
    dst = _layout(jnp.pad(edge_index[1].astype(jnp.int32), (0, pad)),
                  jnp.int32).reshape(-1)
    w = _layout(jnp.pad(edge_weight.astype(jnp.float32), (0, pad)),
                jnp.float32).reshape(-1)
    zeros = jnp.zeros((RPT, DIM), jnp.float32)

    h0 = _li0(x, W0, b0)
    p = _spmm_sc(h0, src, dst, w, zeros)
    h1 = _add(p)
    q = _spmm_sc(h1, src, dst, w, zeros)
    return _li1(q, W1, b1)
